# baseline (speedup 1.0000x reference)
import jax
import jax.numpy as jnp
from jax import lax
from jax.experimental import pallas as pl
from jax.experimental.pallas import tpu as pltpu
from jax.experimental.pallas import tpu_sc as plsc

EMBED_DIM = 64
SEQ_LENGTH = 200
BATCH = 4096

NUM_CORES = 2
NUM_SUBCORES = 16
NUM_WORKERS = NUM_CORES * NUM_SUBCORES
SEQ_PER_WORKER = BATCH // NUM_WORKERS
HALF = SEQ_LENGTH // 2
WIDE = 2 * EMBED_DIM
NBUF = 4
GROUPS = SEQ_PER_WORKER // NBUF


def _body(idx_hbm, word_hbm, pos_hbm, out_hbm, idx_all, pos_sh, rows_v,
          *sems):
    sem_p = sems[0:NBUF]
    sem_g = sems[NBUF:2 * NBUF]
    sem_o = sems[2 * NBUF:3 * NBUF]
    c = lax.axis_index("c")
    s = lax.axis_index("s")
    wid = s * NUM_CORES + c
    base = wid * SEQ_PER_WORKER

    pltpu.sync_copy(idx_hbm.at[wid], idx_all)

    @pl.when(s == 0)
    def _():
        pltpu.sync_copy(pos_hbm, pos_sh)

    plsc.subcore_barrier()

    def group_body(g, carry):
        for b in range(NBUF):
            @pl.when(g > 0)
            def _(b=b):
                for h in range(2):
                    pltpu.make_async_copy(
                        rows_v.at[b, h],
                        out_hbm.at[pl.ds(0, HALF),
                                   pl.ds(h * EMBED_DIM, EMBED_DIM)],
                        sem_o[b]).wait()
            pltpu.async_copy(pos_sh, rows_v.at[b], sem_p[b])
        for b in range(NBUF):
            i = g * NBUF + b
            pltpu.make_async_copy(pos_sh, rows_v.at[b], sem_p[b]).wait()
            for h in range(2):
                pltpu.async_copy(
                    word_hbm.at[idx_all.at[i, h]],
                    rows_v.at[b, h], sem_g[b], add=True)
        for b in range(NBUF):
            i = g * NBUF + b
            for h in range(2):
                pltpu.make_async_copy(
                    word_hbm.at[idx_all.at[i, h]],
                    rows_v.at[b, h], sem_g[b]).wait()
            for h in range(2):
                pltpu.async_copy(
                    rows_v.at[b, h],
                    out_hbm.at[pl.ds((base + i) * HALF, HALF),
                               pl.ds(h * EMBED_DIM, EMBED_DIM)],
                    sem_o[b])
        return carry

    lax.fori_loop(0, GROUPS, group_body, 0)
    for b in range(NBUF):
        for h in range(2):
            pltpu.make_async_copy(
                rows_v.at[b, h],
                out_hbm.at[pl.ds(0, HALF),
                           pl.ds(h * EMBED_DIM, EMBED_DIM)],
                sem_o[b]).wait()


@jax.jit
def kernel(inputs, word_table, pos_table):
    idx = (inputs.astype(jnp.int32)
           .reshape(BATCH, HALF, 2)
           .transpose(0, 2, 1)
           .reshape(NUM_WORKERS, SEQ_PER_WORKER, 2, HALF))
    pos_eo = pos_table.reshape(HALF, 2, EMBED_DIM).transpose(1, 0, 2)
    mesh = plsc.VectorSubcoreMesh(
        core_axis_name="c", subcore_axis_name="s")
    run = pl.kernel(
        _body,
        out_type=jax.ShapeDtypeStruct(
            (BATCH * SEQ_LENGTH * EMBED_DIM // WIDE, WIDE), jnp.float32),
        mesh=mesh,
        scratch_types=[
            pltpu.VMEM((SEQ_PER_WORKER, 2, HALF), jnp.int32),
            pltpu.VMEM_SHARED((2, HALF, EMBED_DIM), jnp.float32),
            pltpu.VMEM((NBUF, 2, HALF, EMBED_DIM), jnp.float32),
        ] + [pltpu.SemaphoreType.DMA] * (3 * NBUF),
        compiler_params=pltpu.CompilerParams(use_tc_tiling_on_sc=False),
    )
    return run(idx, word_table, pos_eo).reshape(
        BATCH, SEQ_LENGTH, EMBED_DIM)
